# baseline (speedup 1.0000x reference)
import jax
import jax.numpy as jnp
from jax.experimental import pallas as pl
from jax.experimental.pallas import tpu as pltpu

N = 10000
IN_FT = 128
OUT_FT = 128
BM = 400


def _fts_kernel(seq_ref, w_ref, fts_ref):
    x = seq_ref[...].astype(jnp.bfloat16)
    w = w_ref[...].astype(jnp.bfloat16)
    fts = jax.lax.dot_general(
        x, w, (((1,), (1,)), ((), ())), preferred_element_type=jnp.float32
    )
    fts_ref[...] = fts.astype(jnp.bfloat16)


def _agg_kernel(adj_ref, fts_ref, bias_ref, a_ref, out_ref):
    a = adj_ref[...].astype(jnp.bfloat16)
    acc = jnp.dot(a, fts_ref[...], preferred_element_type=jnp.float32)
    acc = acc + bias_ref[...]
    alpha = a_ref[0, 0]
    out_ref[...] = jnp.maximum(acc, 0.0) + alpha * jnp.minimum(acc, 0.0)


def kernel(seq, adj, W, bias, prelu_a):
    seq2d = seq.reshape(N, IN_FT)
    adj2d = adj.reshape(N, N)
    bias2d = bias.reshape(1, OUT_FT)
    alpha2d = jnp.asarray(prelu_a, jnp.float32).reshape(1, 1)

    fts = pl.pallas_call(
        _fts_kernel,
        out_shape=jax.ShapeDtypeStruct((N, OUT_FT), jnp.bfloat16),
    )(seq2d, W)

    out = pl.pallas_call(
        _agg_kernel,
        grid=(N // BM,),
        in_specs=[
            pl.BlockSpec((BM, N), lambda i: (i, 0)),
            pl.BlockSpec((N, OUT_FT), lambda i: (0, 0)),
            pl.BlockSpec((1, OUT_FT), lambda i: (0, 0)),
            pl.BlockSpec(memory_space=pltpu.SMEM),
        ],
        out_specs=pl.BlockSpec((BM, OUT_FT), lambda i: (i, 0)),
        out_shape=jax.ShapeDtypeStruct((N, OUT_FT), jnp.float32),
    )(adj2d, fts, bias2d, alpha2d)

    return out.reshape(1, N, OUT_FT)
